# Initial kernel scaffold; baseline (speedup 1.0000x reference)
import jax
import jax.numpy as jnp
from jax.experimental import pallas as pl


def kernel(user, context_item, context_score, global_item, global_score, user_table, item_table, W_agg, domain_id):
    raise NotImplementedError("write your pallas kernel here")


# speedup vs baseline: 13.5853x; 13.5853x over previous
import functools

import jax
import jax.numpy as jnp
from jax import lax
from jax.experimental import pallas as pl
from jax.experimental.pallas import tpu as pltpu
from jax.experimental.pallas import tpu_sc as plsc

LAMBDA_A = 0.7
B, L, D = 16384, 50, 64

NC, NS = 2, 16
NW = NC * NS
SPW = B // NW
C = 8
NCH = SPW // C
ROWS = C * L


def _start_chunk(ctx_flat, item_tbl, idx_buf, rows_buf, sem, base, ch):
    off = (base + ch * C) * L
    pltpu.sync_copy(ctx_flat.at[pl.ds(off, ROWS)], idx_buf)
    pltpu.make_async_copy(item_tbl.at[idx_buf], rows_buf, sem).start()


def _wait_chunk(item_tbl, idx_buf, rows_buf, sem):
    pltpu.make_async_copy(item_tbl.at[idx_buf], rows_buf, sem).wait()


_GATHER_DN = lax.GatherDimensionNumbers(
    offset_dims=(), collapsed_slice_dims=(0,), start_index_map=(0,))


def _lane_perm(v, perm):
    return lax.gather(v, perm[:, None], _GATHER_DN, slice_sizes=(1,),
                      mode=lax.GatherScatterMode.PROMISE_IN_BOUNDS)


def _lane_allsum(v):
    for k in (1, 2, 4, 8):
        perm = lax.iota(jnp.int32, 16) ^ k
        v = v + _lane_perm(v, perm)
    return v


def _compute_chunk(rows_buf, out_buf, ch):
    def sample_body(s, carry):
        r0 = s * L
        V = None
        cnt = jnp.zeros((16,), jnp.float32)
        for l in range(L):
            r = [rows_buf[r0 + l, pl.ds(16 * k, 16)] for k in range(4)]
            if V is None:
                V = list(r)
            else:
                V = [V[k] + r[k] for k in range(4)]
            rs = _lane_allsum((r[0] + r[1]) + (r[2] + r[3]))
            cnt = cnt + jnp.where(rs != 0.0, 1.0, 0.0)
        inv = 1.0 / (cnt + 1e-12)
        o = ch * C + s
        for k in range(4):
            out_buf[o, pl.ds(16 * k, 16)] = V[k] * inv
        return carry

    lax.fori_loop(0, C, sample_body, 0, unroll=False)


def _sc_gather_mean(user, ctx_flat, user_table, item_table):
    mesh = plsc.VectorSubcoreMesh(core_axis_name="c", subcore_axis_name="s")

    @functools.partial(
        pl.kernel,
        mesh=mesh,
        compiler_params=pltpu.CompilerParams(use_tc_tiling_on_sc=False),
        out_type=[
            jax.ShapeDtypeStruct((B, D), jnp.float32),
            jax.ShapeDtypeStruct((B, D), jnp.float32),
        ],
        scratch_types=[
            pltpu.VMEM((ROWS,), jnp.int32),
            pltpu.VMEM((ROWS,), jnp.int32),
            pltpu.VMEM((ROWS, D), jnp.float32),
            pltpu.VMEM((ROWS, D), jnp.float32),
            pltpu.VMEM((SPW,), jnp.int32),
            pltpu.VMEM((SPW, D), jnp.float32),
            pltpu.VMEM((SPW, D), jnp.float32),
            pltpu.SemaphoreType.DMA,
            pltpu.SemaphoreType.DMA,
            pltpu.SemaphoreType.DMA,
        ],
    )
    def sc_kernel(user_h, ctx_h, utab_h, itab_h, out_user_h, out_mean_h,
                  idx_a, idx_b, rows_a, rows_b, uidx, urows, out_buf,
                  sem_a, sem_b, sem_u):
        wid = lax.axis_index("s") * NC + lax.axis_index("c")
        base = wid * SPW

        pltpu.sync_copy(user_h.at[pl.ds(base, SPW)], uidx)
        pltpu.make_async_copy(utab_h.at[uidx], urows, sem_u).start()

        _start_chunk(ctx_h, itab_h, idx_a, rows_a, sem_a, base, 0)

        def outer(t, carry):
            ch0 = 2 * t
            _start_chunk(ctx_h, itab_h, idx_b, rows_b, sem_b, base, ch0 + 1)
            _wait_chunk(itab_h, idx_a, rows_a, sem_a)
            _compute_chunk(rows_a, out_buf, ch0)

            @pl.when(t < NCH // 2 - 1)
            def _():
                _start_chunk(ctx_h, itab_h, idx_a, rows_a, sem_a,
                             base, ch0 + 2)

            _wait_chunk(itab_h, idx_b, rows_b, sem_b)
            _compute_chunk(rows_b, out_buf, ch0 + 1)
            return carry

        lax.fori_loop(0, NCH // 2, outer, 0, unroll=False)

        pltpu.sync_copy(out_buf, out_mean_h.at[pl.ds(base, SPW)])
        pltpu.make_async_copy(utab_h.at[uidx], urows, sem_u).wait()
        pltpu.sync_copy(urows, out_user_h.at[pl.ds(base, SPW)])

    return sc_kernel(user, ctx_flat, user_table, item_table)


def _tc_blend(user_emb, mean, w_t):
    BLK = 2048

    def tc_body(u_ref, m_ref, w_ref, o_ref):
        out = jnp.dot(m_ref[...], w_ref[...],
                      preferred_element_type=jnp.float32)
        o_ref[...] = LAMBDA_A * u_ref[...] + (1.0 - LAMBDA_A) * out

    return pl.pallas_call(
        tc_body,
        grid=(B // BLK,),
        in_specs=[
            pl.BlockSpec((BLK, D), lambda i: (i, 0)),
            pl.BlockSpec((BLK, D), lambda i: (i, 0)),
            pl.BlockSpec((D, D), lambda i: (0, 0)),
        ],
        out_specs=pl.BlockSpec((BLK, D), lambda i: (i, 0)),
        out_shape=jax.ShapeDtypeStruct((B, D), jnp.float32),
    )(user_emb, mean, w_t)


@jax.jit
def _run(user, context_item, user_table, item_table, W_agg):
    ctx_flat = context_item.reshape(B * L).astype(jnp.int32)
    user_i = user.astype(jnp.int32)
    user_emb, mean = _sc_gather_mean(user_i, ctx_flat, user_table, item_table)
    return _tc_blend(user_emb, mean, W_agg.T)


def kernel(user, context_item, context_score, global_item, global_score,
           user_table, item_table, W_agg, domain_id=0):
    return _run(user, context_item, user_table, item_table, W_agg)
